# baseline (speedup 1.0000x reference)
import jax
import jax.numpy as jnp
from jax import lax
from jax.experimental import pallas as pl
from jax.experimental.pallas import tpu as pltpu
from jax.experimental.pallas import tpu_sc as plsc

VOCAB = 100000
EMBED = 128
BATCH = 1024
HIST = 200

_INFO = plsc.get_sparse_core_info()
_NC = _INFO.num_cores
_NS = _INFO.num_subcores
_NW = _NC * _NS

_B = BATCH * HIST
_B_PER_W = _B // _NW
_CHUNK = 128
_N_CHUNKS = _B_PER_W // _CHUNK
_NB = 5
_LA = 2


def _emb_kernel(table_hbm, idx_hbm, out_hbm, idx_all, *bufs_and_sems):
    rows = bufs_and_sems[:_NB]
    gsem = bufs_and_sems[_NB:2 * _NB]
    wsem = bufs_and_sems[2 * _NB:3 * _NB]
    isem = bufs_and_sems[3 * _NB]

    wid = lax.axis_index("s") * _NC + lax.axis_index("c")
    base = wid * _B_PER_W

    _PART = _B_PER_W // 5
    pltpu.sync_copy(idx_hbm.at[pl.ds(base, _PART)], idx_all.at[pl.ds(0, _PART)])
    for p in range(1, 5):
        pltpu.async_copy(idx_hbm.at[pl.ds(base + p * _PART, _PART)],
                         idx_all.at[pl.ds(p * _PART, _PART)], isem)

    def gather(slot, g):
        pltpu.async_copy(
            table_hbm.at[idx_all.at[pl.ds(g * _CHUNK, _CHUNK)]],
            rows[slot], gsem[slot])

    def writeback(slot, g):
        pltpu.async_copy(
            rows[slot], out_hbm.at[pl.ds(base + g * _CHUNK, _CHUNK)],
            wsem[slot])

    gather(0, 0)
    gather(1, 1)

    def body(go, _):
        @pl.when(go == _NB)
        def _():
            for p in range(1, 5):
                pltpu.make_async_copy(
                    idx_hbm.at[pl.ds(0, _B_PER_W // 5)],
                    idx_all.at[pl.ds(0, _B_PER_W // 5)], isem).wait()

        for b in range(_NB):
            g = go + b
            nb = (b + _LA) % _NB

            @pl.when(g + _LA < _N_CHUNKS)
            def _():
                @pl.when(g + _LA >= _NB)
                def _():
                    pltpu.make_async_copy(
                        rows[nb],
                        out_hbm.at[pl.ds(0, _CHUNK)],
                        wsem[nb]).wait()
                gather(nb, g + _LA)

            pltpu.make_async_copy(
                table_hbm.at[idx_all.at[pl.ds(0, _CHUNK)]],
                rows[b], gsem[b]).wait()
            writeback(b, g)
        return ()

    lax.fori_loop(0, _N_CHUNKS // _NB, lambda i, c: body(i * _NB, c), (),
                  unroll=False)

    for b in range(_NB):
        pltpu.make_async_copy(
            rows[b], out_hbm.at[pl.ds(0, _CHUNK)], wsem[b]).wait()


@jax.jit
def _run(x_flat, weight):
    mesh = plsc.VectorSubcoreMesh(core_axis_name="c", subcore_axis_name="s")
    scratch = [pltpu.VMEM((_B_PER_W,), jnp.int32)]
    scratch += [pltpu.VMEM((_CHUNK, EMBED), jnp.float32) for _ in range(_NB)]
    scratch += [pltpu.SemaphoreType.DMA for _ in range(2 * _NB + 1)]
    return pl.kernel(
        _emb_kernel,
        out_type=jax.ShapeDtypeStruct((_B, EMBED), jnp.float32),
        mesh=mesh,
        scratch_types=scratch,
    )(weight, x_flat)


def kernel(x, weight):
    out = _run(x.reshape(_B).astype(jnp.int32), weight)
    return out.reshape(BATCH, HIST, EMBED)


# speedup vs baseline: 1.0011x; 1.0011x over previous
import jax
import jax.numpy as jnp
from jax import lax
from jax.experimental import pallas as pl
from jax.experimental.pallas import tpu as pltpu
from jax.experimental.pallas import tpu_sc as plsc

VOCAB = 100000
EMBED = 128
BATCH = 1024
HIST = 200

_INFO = plsc.get_sparse_core_info()
_NC = _INFO.num_cores
_NS = _INFO.num_subcores
_NW = _NC * _NS

_B = BATCH * HIST
_B_PER_W = _B // _NW
_CHUNK = 128
_N_CHUNKS = _B_PER_W // _CHUNK
_NB = 5
_LA = 3


def _emb_kernel(table_hbm, idx_hbm, out_hbm, idx_all, *bufs_and_sems):
    rows = bufs_and_sems[:_NB]
    gsem = bufs_and_sems[_NB:2 * _NB]
    wsem = bufs_and_sems[2 * _NB:3 * _NB]
    isem = bufs_and_sems[3 * _NB]

    wid = lax.axis_index("s") * _NC + lax.axis_index("c")
    base = wid * _B_PER_W

    _PART = _B_PER_W // 5
    pltpu.sync_copy(idx_hbm.at[pl.ds(base, _PART)], idx_all.at[pl.ds(0, _PART)])
    for p in range(1, 5):
        pltpu.async_copy(idx_hbm.at[pl.ds(base + p * _PART, _PART)],
                         idx_all.at[pl.ds(p * _PART, _PART)], isem)

    def gather(slot, g):
        pltpu.async_copy(
            table_hbm.at[idx_all.at[pl.ds(g * _CHUNK, _CHUNK)]],
            rows[slot], gsem[slot])

    def writeback(slot, g):
        pltpu.async_copy(
            rows[slot], out_hbm.at[pl.ds(base + g * _CHUNK, _CHUNK)],
            wsem[slot])

    for p in range(_LA):
        gather(p, p)

    def body(go, _):
        @pl.when(go == _NB)
        def _():
            for p in range(1, 5):
                pltpu.make_async_copy(
                    idx_hbm.at[pl.ds(0, _B_PER_W // 5)],
                    idx_all.at[pl.ds(0, _B_PER_W // 5)], isem).wait()

        for b in range(_NB):
            g = go + b
            nb = (b + _LA) % _NB

            @pl.when(g + _LA < _N_CHUNKS)
            def _():
                @pl.when(g + _LA >= _NB)
                def _():
                    pltpu.make_async_copy(
                        rows[nb],
                        out_hbm.at[pl.ds(0, _CHUNK)],
                        wsem[nb]).wait()
                gather(nb, g + _LA)

            pltpu.make_async_copy(
                table_hbm.at[idx_all.at[pl.ds(0, _CHUNK)]],
                rows[b], gsem[b]).wait()
            writeback(b, g)
        return ()

    lax.fori_loop(0, _N_CHUNKS // _NB, lambda i, c: body(i * _NB, c), (),
                  unroll=False)

    for b in range(_NB):
        pltpu.make_async_copy(
            rows[b], out_hbm.at[pl.ds(0, _CHUNK)], wsem[b]).wait()


@jax.jit
def _run(x_flat, weight):
    mesh = plsc.VectorSubcoreMesh(core_axis_name="c", subcore_axis_name="s")
    scratch = [pltpu.VMEM((_B_PER_W,), jnp.int32)]
    scratch += [pltpu.VMEM((_CHUNK, EMBED), jnp.float32) for _ in range(_NB)]
    scratch += [pltpu.SemaphoreType.DMA for _ in range(2 * _NB + 1)]
    return pl.kernel(
        _emb_kernel,
        out_type=jax.ShapeDtypeStruct((_B, EMBED), jnp.float32),
        mesh=mesh,
        scratch_types=scratch,
    )(weight, x_flat)


def kernel(x, weight):
    out = _run(x.reshape(_B).astype(jnp.int32), weight)
    return out.reshape(BATCH, HIST, EMBED)
